# Initial kernel scaffold; baseline (speedup 1.0000x reference)
import functools

import jax
import jax.numpy as jnp
from jax import lax
from jax.experimental import pallas as pl
from jax.experimental.pallas import tpu as pltpu
from jax.experimental.pallas import tpu_sc as plsc

D = 128
L = 16
NVR = D // L
NUM_CORES = 2
NUM_SUBCORES = 16
NUM_WORKERS = NUM_CORES * NUM_SUBCORES
CHUNK = 128
EPS = 1e-5


def _ln_kernel_body(n_chunks, rows_per_worker,
                    x_hbm, table_hbm, lnw_hbm, lnb_hbm, out_hbm,
                    idx_v, rows_v, lnw_v, lnb_v, sem):
    wid = lax.axis_index("s") * NUM_CORES + lax.axis_index("c")
    base = wid * rows_per_worker

    pltpu.sync_copy(lnw_hbm, lnw_v)
    pltpu.sync_copy(lnb_hbm, lnb_v)
    lnw = [lnw_v[pl.ds(L * j, L)] for j in range(NVR)]
    lnb = [lnb_v[pl.ds(L * j, L)] for j in range(NVR)]

    inv_d = 1.0 / D

    def chunk_body(i, carry):
        start = base + i * CHUNK
        pltpu.sync_copy(x_hbm.at[pl.ds(start, CHUNK)], idx_v)
        pltpu.async_copy(table_hbm.at[idx_v], rows_v, sem).wait()

        def row_body(r, c):
            v = [rows_v[r, pl.ds(L * j, L)] for j in range(NVR)]
            s = v[0]
            q = v[0] * v[0]
            for j in range(1, NVR):
                s = s + v[j]
                q = q + v[j] * v[j]
            tot = jnp.sum(s)
            totq = jnp.sum(q)
            mean = tot * inv_d
            var = totq * inv_d - mean * mean
            xv = lax.broadcast(var + EPS, (L,))
            ii = plsc.bitcast(xv, jnp.int32)
            ii = 0x5F3759DF - lax.shift_right_logical(ii, 1)
            y = plsc.bitcast(ii, jnp.float32)
            xh = xv * 0.5
            y = y * (1.5 - xh * y * y)
            y = y * (1.5 - xh * y * y)
            y = y * (1.5 - xh * y * y)
            for j in range(NVR):
                rows_v[r, pl.ds(L * j, L)] = (v[j] - mean) * y * lnw[j] + lnb[j]
            return c

        lax.fori_loop(0, CHUNK, row_body, 0, unroll=2)
        pltpu.sync_copy(rows_v, out_hbm.at[pl.ds(start, CHUNK)])
        return carry

    lax.fori_loop(0, n_chunks, chunk_body, 0)


def kernel(x, table, ln_w, ln_b):
    b, s = x.shape
    total = b * s
    assert total % (NUM_WORKERS * CHUNK) == 0
    rows_per_worker = total // NUM_WORKERS
    n_chunks = rows_per_worker // CHUNK

    x_flat = x.reshape(total)

    mesh = plsc.VectorSubcoreMesh(core_axis_name="c", subcore_axis_name="s")
    fn = pl.kernel(
        functools.partial(_ln_kernel_body, n_chunks, rows_per_worker),
        out_type=jax.ShapeDtypeStruct((total, D), jnp.float32),
        mesh=mesh,
        scratch_types=[
            pltpu.VMEM((CHUNK,), jnp.int32),
            pltpu.VMEM((CHUNK, D), jnp.float32),
            pltpu.VMEM((D,), jnp.float32),
            pltpu.VMEM((D,), jnp.float32),
            pltpu.SemaphoreType.DMA,
        ],
    )
    out = fn(x_flat, table, ln_w, ln_b)
    return out.reshape(b, s, D)


# speedup vs baseline: 2.3570x; 2.3570x over previous
import functools

import jax
import jax.numpy as jnp
from jax import lax
from jax.experimental import pallas as pl
from jax.experimental.pallas import tpu as pltpu
from jax.experimental.pallas import tpu_sc as plsc

D = 128
L = 16
NVR = D // L
NUM_CORES = 2
NUM_SUBCORES = 16
NUM_WORKERS = NUM_CORES * NUM_SUBCORES
CHUNK = 128
EPS = 1e-5

_GATHER_DNUMS = lax.GatherDimensionNumbers(
    offset_dims=(), collapsed_slice_dims=(0,), start_index_map=(0,))


def _vperm(v, idx2d):
    return lax.gather(v, idx2d, _GATHER_DNUMS, slice_sizes=(1,),
                      mode=lax.GatherScatterMode.PROMISE_IN_BOUNDS)


def _ln_kernel_body(n_chunks, rows_per_worker,
                    x_hbm, table_hbm, lnw_hbm, lnb_hbm, out_hbm,
                    idx_v, rows_v, lnw_v, lnb_v, sem):
    wid = lax.axis_index("s") * NUM_CORES + lax.axis_index("c")
    base = wid * rows_per_worker

    pltpu.sync_copy(lnw_hbm, lnw_v)
    pltpu.sync_copy(lnb_hbm, lnb_v)
    lnw = [lnw_v[pl.ds(L * j, L)] for j in range(NVR)]
    lnb = [lnb_v[pl.ds(L * j, L)] for j in range(NVR)]

    inv_d = 1.0 / D
    iota = lax.iota(jnp.int32, L)
    bfly = [(iota ^ (1 << k)).reshape(L, 1) for k in range(4)]

    def chunk_body(i, carry):
        start = base + i * CHUNK
        pltpu.sync_copy(x_hbm.at[pl.ds(start, CHUNK)], idx_v)
        pltpu.async_copy(table_hbm.at[idx_v], rows_v, sem).wait()

        def row_body(r, c):
            v = [rows_v[r, pl.ds(L * j, L)] for j in range(NVR)]
            s = v[0]
            q = v[0] * v[0]
            for j in range(1, NVR):
                s = s + v[j]
                q = q + v[j] * v[j]
            for idx in bfly:
                s = s + _vperm(s, idx)
                q = q + _vperm(q, idx)
            mean = s * inv_d
            var = q * inv_d - mean * mean
            xv = var + EPS
            ii = plsc.bitcast(xv, jnp.int32)
            ii = 0x5F3759DF - lax.shift_right_logical(ii, 1)
            y = plsc.bitcast(ii, jnp.float32)
            xh = xv * 0.5
            y = y * (1.5 - xh * y * y)
            y = y * (1.5 - xh * y * y)
            y = y * (1.5 - xh * y * y)
            for j in range(NVR):
                rows_v[r, pl.ds(L * j, L)] = (v[j] - mean) * y * lnw[j] + lnb[j]
            return c

        lax.fori_loop(0, CHUNK, row_body, 0, unroll=2)
        pltpu.sync_copy(rows_v, out_hbm.at[pl.ds(start, CHUNK)])
        return carry

    lax.fori_loop(0, n_chunks, chunk_body, 0)


def kernel(x, table, ln_w, ln_b):
    b, s = x.shape
    total = b * s
    assert total % (NUM_WORKERS * CHUNK) == 0
    rows_per_worker = total // NUM_WORKERS
    n_chunks = rows_per_worker // CHUNK

    x_flat = x.reshape(total)

    mesh = plsc.VectorSubcoreMesh(
        core_axis_name="c", subcore_axis_name="s",
        num_cores=NUM_CORES, num_subcores=NUM_SUBCORES)
    fn = pl.kernel(
        functools.partial(_ln_kernel_body, n_chunks, rows_per_worker),
        out_type=jax.ShapeDtypeStruct((total, D), jnp.float32),
        mesh=mesh,
        scratch_types=[
            pltpu.VMEM((CHUNK,), jnp.int32),
            pltpu.VMEM((CHUNK, D), jnp.float32),
            pltpu.VMEM((D,), jnp.float32),
            pltpu.VMEM((D,), jnp.float32),
            pltpu.SemaphoreType.DMA,
        ],
        compiler_params=pltpu.CompilerParams(needs_layout_passes=False),
    )
    out = fn(x_flat, table, ln_w, ln_b)
    return out.reshape(b, s, D)
